# baseline (speedup 1.0000x reference)
import functools

import jax
import jax.numpy as jnp
from jax import lax
from jax.experimental import pallas as pl
from jax.experimental.pallas import tpu as pltpu
from jax.experimental.pallas import tpu_sc as plsc

_VOCAB = 100000
_D = 128
_BATCH = 1024
_SEQ = 200
_B = _BATCH * _SEQ

_NC = 2
_NS = 16
_NW = _NC * _NS
_BPW = _B // _NW
_CH = 80
_NCHUNK = _BPW // _CH

_NBUF = 8
_PRE = 4

_mesh = plsc.VectorSubcoreMesh(core_axis_name="c", subcore_axis_name="s")


@functools.partial(
    pl.kernel,
    mesh=_mesh,
    out_type=jax.ShapeDtypeStruct((_B, _D), jnp.float32),
    scratch_types=(
        [pltpu.VMEM((_NCHUNK, _CH), jnp.int32)]
        + [pltpu.VMEM((_CH, _D), jnp.float32)] * _NBUF
        + [pltpu.SemaphoreType.DMA] * (2 * _NBUF)
    ),
)
def _sc_gather(table_hbm, idx_hbm, out_hbm, idx_v, *bufs):
    rows = bufs[:_NBUF]
    gsem = bufs[_NBUF:2 * _NBUF]
    wsem = bufs[2 * _NBUF:]

    wid = lax.axis_index("s") * _NC + lax.axis_index("c")
    base = wid * _BPW
    pltpu.sync_copy(idx_hbm.at[wid], idx_v)

    def g_copy(c, b):
        return pltpu.make_async_copy(table_hbm.at[idx_v.at[c]], rows[b], gsem[b])

    def w_copy(c, b):
        return pltpu.make_async_copy(
            rows[b], out_hbm.at[pl.ds(base + c * _CH, _CH)], wsem[b])

    def stage(c, b):
        nb = (b + _PRE) % _NBUF
        g_copy(c, b).wait()
        w_copy(c, b).start()
        w_copy(c + _PRE - _NBUF, nb).wait()
        g_copy(c + _PRE, nb).start()

    lo = _NBUF - _PRE
    for c in range(_PRE):
        g_copy(c, c % _NBUF).start()
    for c in range(lo):
        b = c % _NBUF
        g_copy(c, b).wait()
        w_copy(c, b).start()
        g_copy(c + _PRE, (c + _PRE) % _NBUF).start()

    hi = _NCHUNK - _PRE
    k = (hi - lo) // _NBUF

    def step(o, carry):
        c0 = lo + o * _NBUF
        for cc in range(_NBUF):
            stage(c0 + cc, (lo + cc) % _NBUF)
        return carry

    lax.fori_loop(0, k, step, 0)

    for c in range(lo + k * _NBUF, hi):
        stage(c, c % _NBUF)
    for c in range(hi, _NCHUNK):
        b = c % _NBUF
        g_copy(c, b).wait()
        w_copy(c, b).start()
    for c in range(_NCHUNK - _NBUF, _NCHUNK):
        w_copy(c, c % _NBUF).wait()


def kernel(indices, table):
    idx3 = indices.reshape(_NW, _NCHUNK, _CH)
    out = _sc_gather(table, idx3)
    return out.reshape(_BATCH, _SEQ, _D)


# speedup vs baseline: 1.0076x; 1.0076x over previous
import functools

import jax
import jax.numpy as jnp
from jax import lax
from jax.experimental import pallas as pl
from jax.experimental.pallas import tpu as pltpu
from jax.experimental.pallas import tpu_sc as plsc

_VOCAB = 100000
_D = 128
_BATCH = 1024
_SEQ = 200
_B = _BATCH * _SEQ

_NC = 2
_NS = 16
_NW = _NC * _NS
_RPW = _BATCH // _NW
_BPW = _RPW * _SEQ
_SZ = (128, 72)
_OFF = (0, 128)
_NCHUNK = 2 * _RPW

_NBUF = 8
_PRE = 4

_mesh = plsc.VectorSubcoreMesh(core_axis_name="c", subcore_axis_name="s")


@functools.partial(
    pl.kernel,
    mesh=_mesh,
    out_type=jax.ShapeDtypeStruct((_B, _D), jnp.float32),
    scratch_types=(
        [pltpu.VMEM((_RPW, _SZ[0]), jnp.int32),
         pltpu.VMEM((_RPW, _SZ[1]), jnp.int32)]
        + [pltpu.VMEM((_SZ[b % 2], _D), jnp.float32) for b in range(_NBUF)]
        + [pltpu.SemaphoreType.DMA] * (2 * _NBUF)
    ),
)
def _sc_gather(table_hbm, idx_hbm, out_hbm, idx_a, idx_b, *bufs):
    rows = bufs[:_NBUF]
    gsem = bufs[_NBUF:2 * _NBUF]
    wsem = bufs[2 * _NBUF:]

    wid = lax.axis_index("s") * _NC + lax.axis_index("c")
    base = wid * _BPW
    pltpu.sync_copy(
        idx_hbm.at[pl.ds(wid * _RPW, _RPW), pl.ds(0, _SZ[0])], idx_a)
    pltpu.sync_copy(
        idx_hbm.at[pl.ds(wid * _RPW, _RPW), pl.ds(_OFF[1], _SZ[1])], idx_b)
    idx_v = (idx_a, idx_b)

    def g_copy(r, p, b):
        idx_list = idx_v[p].at[r]
        return pltpu.make_async_copy(table_hbm.at[idx_list], rows[b], gsem[b])

    def w_copy(r, p, b):
        dst = out_hbm.at[pl.ds(base + r * _SEQ + _OFF[p], _SZ[p])]
        return pltpu.make_async_copy(rows[b], dst, wsem[b])

    def stage(r, p, b):
        nb = (b + _PRE) % _NBUF
        g_copy(r, p, b).wait()
        w_copy(r, p, b).start()
        w_copy(r + (_PRE - _NBUF) // 2, p, nb).wait()
        g_copy(r + _PRE // 2, p, nb).start()

    lo = _NBUF - _PRE
    for c in range(_PRE):
        g_copy(c // 2, c % 2, c % _NBUF).start()
    for c in range(lo):
        b = c % _NBUF
        g_copy(c // 2, c % 2, b).wait()
        w_copy(c // 2, c % 2, b).start()
        g_copy((c + _PRE) // 2, c % 2, (c + _PRE) % _NBUF).start()

    hi = _NCHUNK - _PRE
    k = (hi - lo) // _NBUF

    def step(o, carry):
        r0 = (lo + o * _NBUF) // 2
        for cc in range(_NBUF):
            stage(r0 + (lo + cc) // 2 - lo // 2, cc % 2, (lo + cc) % _NBUF)
        return carry

    lax.fori_loop(0, k, step, 0)

    for c in range(lo + k * _NBUF, hi):
        stage(c // 2, c % 2, c % _NBUF)
    for c in range(hi, _NCHUNK):
        b = c % _NBUF
        g_copy(c // 2, c % 2, b).wait()
        w_copy(c // 2, c % 2, b).start()
    for c in range(_NCHUNK - _NBUF, _NCHUNK):
        w_copy(c // 2, c % 2, c % _NBUF).wait()


def kernel(indices, table):
    out = _sc_gather(table, indices)
    return out.reshape(_BATCH, _SEQ, _D)


# speedup vs baseline: 1.0090x; 1.0013x over previous
import functools

import jax
import jax.numpy as jnp
from jax import lax
from jax.experimental import pallas as pl
from jax.experimental.pallas import tpu as pltpu
from jax.experimental.pallas import tpu_sc as plsc

_VOCAB = 100000
_D = 128
_BATCH = 1024
_SEQ = 200
_B = _BATCH * _SEQ

_NC = 2
_NS = 16
_NW = _NC * _NS
_RPW = _BATCH // _NW
_BPW = _RPW * _SEQ
_SZ = (128, 72)
_OFF = (0, 128)
_NCHUNK = 2 * _RPW

_NBUF = 8
_PRE = 4

_mesh = plsc.VectorSubcoreMesh(core_axis_name="c", subcore_axis_name="s")


@functools.partial(
    pl.kernel,
    mesh=_mesh,
    compiler_params=pltpu.CompilerParams(use_tc_tiling_on_sc=True),
    out_type=jax.ShapeDtypeStruct((_B, _D), jnp.float32),
    scratch_types=(
        [pltpu.VMEM((_RPW, _SZ[0]), jnp.int32),
         pltpu.VMEM((_RPW, _SZ[1]), jnp.int32)]
        + [pltpu.VMEM((_SZ[b % 2], _D), jnp.float32) for b in range(_NBUF)]
        + [pltpu.SemaphoreType.DMA] * (2 * _NBUF)
    ),
)
def _sc_gather(table_hbm, idx_hbm, out_hbm, idx_a, idx_b, *bufs):
    rows = bufs[:_NBUF]
    gsem = bufs[_NBUF:2 * _NBUF]
    wsem = bufs[2 * _NBUF:]

    wid = lax.axis_index("s") * _NC + lax.axis_index("c")
    base = wid * _BPW
    pltpu.sync_copy(
        idx_hbm.at[pl.ds(wid * _RPW, _RPW), pl.ds(0, _SZ[0])], idx_a)
    pltpu.sync_copy(
        idx_hbm.at[pl.ds(wid * _RPW, _RPW), pl.ds(_OFF[1], _SZ[1])], idx_b)
    idx_v = (idx_a, idx_b)

    def g_copy(r, p, b):
        idx_list = idx_v[p].at[r]
        return pltpu.make_async_copy(table_hbm.at[idx_list], rows[b], gsem[b])

    def w_copy(r, p, b):
        dst = out_hbm.at[pl.ds(base + r * _SEQ + _OFF[p], _SZ[p])]
        return pltpu.make_async_copy(rows[b], dst, wsem[b])

    def stage(r, p, b):
        nb = (b + _PRE) % _NBUF
        g_copy(r, p, b).wait()
        w_copy(r, p, b).start()
        w_copy(r + (_PRE - _NBUF) // 2, p, nb).wait()
        g_copy(r + _PRE // 2, p, nb).start()

    lo = _NBUF - _PRE
    for c in range(_PRE):
        g_copy(c // 2, c % 2, c % _NBUF).start()
    for c in range(lo):
        b = c % _NBUF
        g_copy(c // 2, c % 2, b).wait()
        w_copy(c // 2, c % 2, b).start()
        g_copy((c + _PRE) // 2, c % 2, (c + _PRE) % _NBUF).start()

    hi = _NCHUNK - _PRE
    k = (hi - lo) // _NBUF

    def step(o, carry):
        r0 = (lo + o * _NBUF) // 2
        for cc in range(_NBUF):
            stage(r0 + (lo + cc) // 2 - lo // 2, cc % 2, (lo + cc) % _NBUF)
        return carry

    lax.fori_loop(0, k, step, 0)

    for c in range(lo + k * _NBUF, hi):
        stage(c // 2, c % 2, c % _NBUF)
    for c in range(hi, _NCHUNK):
        b = c % _NBUF
        g_copy(c // 2, c % 2, b).wait()
        w_copy(c // 2, c % 2, b).start()
    for c in range(_NCHUNK - _NBUF, _NCHUNK):
        w_copy(c // 2, c % 2, c % _NBUF).wait()


def kernel(indices, table):
    out = _sc_gather(table, indices)
    return out.reshape(_BATCH, _SEQ, _D)


# speedup vs baseline: 1.0094x; 1.0005x over previous
import functools

import jax
import jax.numpy as jnp
from jax import lax
from jax.experimental import pallas as pl
from jax.experimental.pallas import tpu as pltpu
from jax.experimental.pallas import tpu_sc as plsc

_VOCAB = 100000
_D = 128
_BATCH = 1024
_SEQ = 200
_B = _BATCH * _SEQ

_NC = 2
_NS = 16
_NW = _NC * _NS
_RPW = _BATCH // _NW
_BPW = _RPW * _SEQ
_SZ = (128, 72)
_OFF = (0, 128)
_NCHUNK = 2 * _RPW

_NBUF = 8
_PRE = 4

_mesh = plsc.VectorSubcoreMesh(core_axis_name="c", subcore_axis_name="s")


@functools.partial(
    pl.kernel,
    mesh=_mesh,
    out_type=jax.ShapeDtypeStruct((_B, _D), jnp.float32),
    scratch_types=(
        [pltpu.VMEM((_RPW, _SZ[0]), jnp.int32),
         pltpu.VMEM((_RPW, _SZ[1]), jnp.int32)]
        + [pltpu.VMEM((_SZ[b % 2], _D), jnp.float32) for b in range(_NBUF)]
        + [pltpu.SemaphoreType.DMA] * (2 * _NBUF)
    ),
)
def _sc_gather(table_hbm, idx_hbm, out_hbm, idx_a, idx_b, *bufs):
    rows = bufs[:_NBUF]
    gsem = bufs[_NBUF:2 * _NBUF]
    wsem = bufs[2 * _NBUF:]

    wid = lax.axis_index("s") * _NC + lax.axis_index("c")
    base = wid * _BPW
    pltpu.sync_copy(
        idx_hbm.at[pl.ds(wid * _RPW, _RPW), pl.ds(0, _SZ[0])], idx_a)
    pltpu.sync_copy(
        idx_hbm.at[pl.ds(wid * _RPW, _RPW), pl.ds(_OFF[1], _SZ[1])], idx_b)
    idx_v = (idx_a, idx_b)

    def g_copy(r, p, b):
        idx_list = idx_v[p].at[r]
        return pltpu.make_async_copy(table_hbm.at[idx_list], rows[b], gsem[b])

    def w_copy(r, p, b):
        dst = out_hbm.at[pl.ds(base + r * _SEQ + _OFF[p], _SZ[p])]
        return pltpu.make_async_copy(rows[b], dst, wsem[b])

    def stage(r, p, b):
        nb = (b + _PRE) % _NBUF
        g_copy(r, p, b).wait()
        w_copy(r, p, b).start()
        w_copy(r + (_PRE - _NBUF) // 2, p, nb).wait()
        g_copy(r + _PRE // 2, p, nb).start()

    lo = _NBUF - _PRE
    for c in range(_PRE):
        g_copy(c // 2, c % 2, c % _NBUF).start()
    for c in range(lo):
        b = c % _NBUF
        g_copy(c // 2, c % 2, b).wait()
        w_copy(c // 2, c % 2, b).start()
        g_copy((c + _PRE) // 2, c % 2, (c + _PRE) % _NBUF).start()

    hi = _NCHUNK - _PRE
    k = (hi - lo) // _NBUF

    def step(o, carry):
        r0 = (lo + o * _NBUF) // 2
        for cc in range(_NBUF):
            stage(r0 + (lo + cc) // 2 - lo // 2, cc % 2, (lo + cc) % _NBUF)
        return carry

    lax.fori_loop(0, k, step, 0)

    for c in range(lo + k * _NBUF, hi):
        stage(c // 2, c % 2, c % _NBUF)
    for c in range(hi, _NCHUNK):
        b = c % _NBUF
        g_copy(c // 2, c % 2, b).wait()
        w_copy(c // 2, c % 2, b).start()
    for c in range(_NCHUNK - _NBUF, _NCHUNK):
        w_copy(c // 2, c % 2, c % _NBUF).wait()


def kernel(indices, table):
    out = _sc_gather(table, indices)
    return out.reshape(_BATCH, _SEQ, _D)
